# baseline (speedup 1.0000x reference)
import jax
import jax.numpy as jnp
from jax import lax
from jax.experimental import pallas as pl
from jax.experimental.pallas import tpu as pltpu
from jax.experimental.pallas import tpu_sc as plsc

BATCH = 4096
DIM = 128
LANES = 16
NUM_WORKERS = 32
B_PER_W = BATCH // NUM_WORKERS
CHUNKS = DIM // LANES


def _body(
    session_idx_hbm,
    subject_idx_hbm,
    session_table_hbm,
    subject_table_hbm,
    session_flag_hbm,
    subject_flag_hbm,
    out_hbm,
    idx_s_v,
    idx_u_v,
    rows_s_v,
    rows_u_v,
    flag_s_v,
    flag_u_v,
    sem_is,
    sem_iu,
    sem_flag,
    sem_s,
    sem_u,
    sem_out,
):
    wid = lax.axis_index("c") * 16 + lax.axis_index("s")
    base = wid * B_PER_W

    cp_is = pltpu.async_copy(session_idx_hbm.at[pl.ds(base, B_PER_W)], idx_s_v, sem_is)
    cp_iu = pltpu.async_copy(subject_idx_hbm.at[pl.ds(base, B_PER_W)], idx_u_v, sem_iu)
    cp_fs = pltpu.async_copy(session_flag_hbm, flag_s_v, sem_flag)
    cp_fu = pltpu.async_copy(subject_flag_hbm, flag_u_v, sem_flag)

    cp_is.wait()
    g_s = pltpu.async_copy(session_table_hbm.at[idx_s_v], rows_s_v, sem_s)
    cp_iu.wait()
    g_u = pltpu.async_copy(subject_table_hbm.at[idx_u_v], rows_u_v, sem_u)

    cp_fs.wait()
    cp_fu.wait()
    fl_s = [flag_s_v[pl.ds(c * LANES, LANES)] for c in range(CHUNKS)]
    fl_u = [flag_u_v[pl.ds(c * LANES, LANES)] for c in range(CHUNKS)]

    def add_flags(rows_v, fl):
        @plsc.parallel_loop(0, B_PER_W, unroll=1)
        def _(r):
            for c in range(CHUNKS):
                sl = pl.ds(c * LANES, LANES)
                rows_v[r, sl] = rows_v[r, sl] + fl[c]

    g_s.wait()
    add_flags(rows_s_v, fl_s)
    put_s = pltpu.async_copy(rows_s_v, out_hbm.at[pl.ds(base, B_PER_W), 0], sem_out)

    g_u.wait()
    add_flags(rows_u_v, fl_u)
    put_u = pltpu.async_copy(rows_u_v, out_hbm.at[pl.ds(base, B_PER_W), 1], sem_out)

    put_s.wait()
    put_u.wait()


@jax.jit
def kernel(session_idx, subject_idx, session_table, subject_table, session_flag, subject_flag):
    mesh = plsc.VectorSubcoreMesh(core_axis_name="c", subcore_axis_name="s")
    run = pl.kernel(
        _body,
        out_type=jax.ShapeDtypeStruct((BATCH, 2, DIM), jnp.float32),
        mesh=mesh,
        scratch_types=[
            pltpu.VMEM((B_PER_W,), jnp.int32),
            pltpu.VMEM((B_PER_W,), jnp.int32),
            pltpu.VMEM((B_PER_W, DIM), jnp.float32),
            pltpu.VMEM((B_PER_W, DIM), jnp.float32),
            pltpu.VMEM((DIM,), jnp.float32),
            pltpu.VMEM((DIM,), jnp.float32),
            pltpu.SemaphoreType.DMA,
            pltpu.SemaphoreType.DMA,
            pltpu.SemaphoreType.DMA,
            pltpu.SemaphoreType.DMA,
            pltpu.SemaphoreType.DMA,
            pltpu.SemaphoreType.DMA,
        ],
    )
    return run(
        session_idx.astype(jnp.int32),
        subject_idx.astype(jnp.int32),
        session_table,
        subject_table,
        session_flag,
        subject_flag,
    )


# speedup vs baseline: 1.0087x; 1.0087x over previous
import jax
import jax.numpy as jnp
from jax import lax
from jax.experimental import pallas as pl
from jax.experimental.pallas import tpu as pltpu
from jax.experimental.pallas import tpu_sc as plsc

BATCH = 4096
DIM = 128
LANES = 16
NUM_WORKERS = 32
B_PER_W = BATCH // NUM_WORKERS
CHUNKS = DIM // LANES


def _body(
    session_idx_hbm,
    subject_idx_hbm,
    session_table_hbm,
    subject_table_hbm,
    session_flag_hbm,
    subject_flag_hbm,
    out_hbm,
    idx_s_v,
    idx_u_v,
    rows_s_v,
    rows_u_v,
    flag_s_v,
    flag_u_v,
    sem_is,
    sem_iu,
    sem_flag,
    sem_s,
    sem_u,
    sem_out,
):
    wid = lax.axis_index("s") * 2 + lax.axis_index("c")
    base = wid * B_PER_W

    cp_is = pltpu.async_copy(session_idx_hbm.at[pl.ds(base, B_PER_W)], idx_s_v, sem_is)
    cp_iu = pltpu.async_copy(subject_idx_hbm.at[pl.ds(base, B_PER_W)], idx_u_v, sem_iu)
    cp_fs = pltpu.async_copy(session_flag_hbm, flag_s_v, sem_flag)
    cp_fu = pltpu.async_copy(subject_flag_hbm, flag_u_v, sem_flag)

    cp_is.wait()
    g_s = pltpu.async_copy(session_table_hbm.at[idx_s_v], rows_s_v, sem_s)
    cp_iu.wait()
    g_u = pltpu.async_copy(subject_table_hbm.at[idx_u_v], rows_u_v, sem_u)

    cp_fs.wait()
    cp_fu.wait()
    fl_s = [flag_s_v[pl.ds(c * LANES, LANES)] for c in range(CHUNKS)]
    fl_u = [flag_u_v[pl.ds(c * LANES, LANES)] for c in range(CHUNKS)]

    def add_flags(rows_v, fl):
        @plsc.parallel_loop(0, B_PER_W, unroll=1)
        def _(r):
            for c in range(CHUNKS):
                sl = pl.ds(c * LANES, LANES)
                rows_v[r, sl] = rows_v[r, sl] + fl[c]

    g_s.wait()
    add_flags(rows_s_v, fl_s)
    put_s = pltpu.async_copy(rows_s_v, out_hbm.at[pl.ds(base, B_PER_W), 0], sem_out)

    g_u.wait()
    add_flags(rows_u_v, fl_u)
    put_u = pltpu.async_copy(rows_u_v, out_hbm.at[pl.ds(base, B_PER_W), 1], sem_out)

    put_s.wait()
    put_u.wait()


@jax.jit
def kernel(session_idx, subject_idx, session_table, subject_table, session_flag, subject_flag):
    mesh = plsc.VectorSubcoreMesh(core_axis_name="c", subcore_axis_name="s")
    run = pl.kernel(
        _body,
        out_type=jax.ShapeDtypeStruct((BATCH, 2, DIM), jnp.float32),
        mesh=mesh,
        scratch_types=[
            pltpu.VMEM((B_PER_W,), jnp.int32),
            pltpu.VMEM((B_PER_W,), jnp.int32),
            pltpu.VMEM((B_PER_W, DIM), jnp.float32),
            pltpu.VMEM((B_PER_W, DIM), jnp.float32),
            pltpu.VMEM((DIM,), jnp.float32),
            pltpu.VMEM((DIM,), jnp.float32),
            pltpu.SemaphoreType.DMA,
            pltpu.SemaphoreType.DMA,
            pltpu.SemaphoreType.DMA,
            pltpu.SemaphoreType.DMA,
            pltpu.SemaphoreType.DMA,
            pltpu.SemaphoreType.DMA,
        ],
    )
    return run(
        session_idx.astype(jnp.int32),
        subject_idx.astype(jnp.int32),
        session_table,
        subject_table,
        session_flag,
        subject_flag,
    )
